# baseline (speedup 1.0000x reference)
import functools

import jax
import jax.numpy as jnp
from jax import lax
from jax.experimental import pallas as pl
from jax.experimental.pallas import tpu as pltpu
from jax.experimental.pallas import tpu_sc as plsc

BATCH = 1024
SEQ = 200
DIM = 64
HALF = SEQ // 2
LANES = 16
NUM_CORES = 2
NUM_SUBCORES = 16
NW = NUM_CORES * NUM_SUBCORES
ROWS_PER_W = BATCH // NW
CHUNK = 2
NCHUNK = ROWS_PER_W // CHUNK
NBUF = 4
G = CHUNK * 2


def _body(idx_hbm, tok_hbm, pos_hbm, out_hbm,
          idx_v, pos_v, bufs, gsems, wsems):
    wid = lax.axis_index("s") * NUM_CORES + lax.axis_index("c")
    gbase = wid * NCHUNK

    pltpu.sync_copy(pos_hbm, pos_v)
    pltpu.sync_copy(idx_hbm.at[pl.ds(gbase, NCHUNK)], idx_v)

    def fire_gather(chunk, b):
        return [pltpu.async_copy(
            tok_hbm.at[idx_v.at[chunk, j]], bufs[b].at[j], gsems[b])
            for j in range(G)]

    gcps = [None] * NBUF
    wcps = [None] * NBUF
    for k in range(NBUF - 1):
        gcps[k] = fire_gather(k, k)

    for chunk in range(NCHUNK):
        b = chunk % NBUF
        fb = (chunk + NBUF - 1) % NBUF
        if chunk + NBUF - 1 < NCHUNK:
            if wcps[fb] is not None:
                wcps[fb].wait()
                wcps[fb] = None
            gcps[fb] = fire_gather(chunk + NBUF - 1, fb)
        for cp in gcps[b]:
            cp.wait()

        for j in range(G):
            def add_body(r, c2, _j=j, _b=b):
                for c in range(DIM // LANES):
                    v = pos_v[_j % 2, r, pl.ds(c * LANES, LANES)]
                    plsc.addupdate(
                        bufs[_b].at[_j, r, pl.ds(c * LANES, LANES)], v)
                return c2
            lax.fori_loop(0, HALF, add_body, 0, unroll=4)

        wcps[b] = pltpu.async_copy(bufs[b], out_hbm.at[gbase + chunk],
                                   wsems[b])

    for b in range(NBUF):
        if wcps[b] is not None:
            wcps[b].wait()


@jax.jit
def _run(idx, token_table, pos_rep):
    mesh = plsc.VectorSubcoreMesh(core_axis_name="c", subcore_axis_name="s")
    f = functools.partial(
        pl.kernel,
        out_type=jax.ShapeDtypeStruct((NW * NCHUNK, G, HALF, DIM),
                                      jnp.float32),
        mesh=mesh,
        scratch_types=[
            pltpu.VMEM((NCHUNK, G, HALF), jnp.int32),
            pltpu.VMEM((2, HALF, DIM), jnp.float32),
            [pltpu.VMEM((G, HALF, DIM), jnp.float32)] * NBUF,
            [pltpu.SemaphoreType.DMA] * NBUF,
            [pltpu.SemaphoreType.DMA] * NBUF,
        ],
        compiler_params=pltpu.CompilerParams(use_tc_tiling_on_sc=False),
    )(_body)
    return f(idx, token_table, pos_rep)


def kernel(inputs, token_table, position_table):
    idx = inputs.reshape(NW * NCHUNK, G, HALF).astype(jnp.int32)
    pos2 = position_table.reshape(2, HALF, DIM)
    out = _run(idx, token_table, pos2)
    return out.reshape(BATCH, SEQ, DIM)
